# Initial kernel scaffold; baseline (speedup 1.0000x reference)
import jax
import jax.numpy as jnp
from jax.experimental import pallas as pl


def kernel(X, params):
    raise NotImplementedError("write your pallas kernel here")


# speedup vs baseline: 2.6467x; 2.6467x over previous
import functools

import jax
import jax.numpy as jnp
from jax import lax
from jax.experimental import pallas as pl
from jax.experimental.pallas import tpu as pltpu
from jax.experimental.pallas import tpu_sc as plsc

B = 4096
D = 26
AR = 16
OD = 32
TAB = 2 ** AR
NP = B * D
NW = 32
PPW = NP // NW
CH = 64
NCH = PPW // CH
IDX_ROWS = CH * AR // 128

def _hoa_body(x_hbm, tab_hbm, out_hbm, x_v, coef_v, idx_v, rows_v, out_v,
              sa_v, sem):
  wid = lax.axis_index("s") * 2 + lax.axis_index("c")
  base = wid * PPW
  iota = lax.iota(jnp.int32, AR)
  prev_idx = jnp.maximum(iota - 1, 0)
  lane0 = iota == 0
  ones = (iota >= 0).astype(jnp.int32)

  def chunk_body(c, carry):
    pair0 = base + c * CH
    pltpu.sync_copy(x_hbm.at[pl.ds(pair0, CH)], x_v)

    def build(p, carry2):
      a = x_v[p]
      sa, ind = plsc.sort_key_val(a, iota)
      sa_v[...] = sa
      prev = plsc.load_gather(sa_v, [prev_idx])
      coef = jnp.where(lane0, sa, sa - prev)
      sh = jnp.left_shift(ones, ind)
      cs = plsc.cumsum(sh)
      field = (pair0 + p) % D
      gidx = (field * TAB + 65535 - cs) + sh
      idx_v[p // 8, pl.ds((p % 8) * AR, AR)] = gidx
      coef_v[p] = coef
      return carry2

    lax.fori_loop(0, CH, build, 0)

    copies = [
        pltpu.async_copy(
            tab_hbm.at[idx_v.at[g]],
            rows_v.at[pl.ds(g * 128, 128)],
            sem,
        )
        for g in range(IDX_ROWS)
    ]
    for cp in copies:
      cp.wait()

    def reduce(p, carry2):
      acc0 = jnp.zeros((16,), jnp.float32)
      acc1 = jnp.zeros((16,), jnp.float32)
      pvec = jnp.full((16,), p, jnp.int32)
      r = p * AR
      for k in range(AR):
        cb = plsc.load_gather(coef_v, [pvec, jnp.full((16,), k, jnp.int32)])
        acc0 = acc0 + cb * rows_v[r + k, 0:16]
        acc1 = acc1 + cb * rows_v[r + k, 16:32]
      out_v[p, 0:16] = acc0
      out_v[p, 16:32] = acc1
      return carry2

    lax.fori_loop(0, CH, reduce, 0)
    pltpu.sync_copy(out_v, out_hbm.at[pl.ds(pair0, CH)])
    return carry

  lax.fori_loop(0, NCH, chunk_body, 0)


@jax.jit
def _hoa(xf, tab):
  mesh = plsc.VectorSubcoreMesh(core_axis_name="c", subcore_axis_name="s")
  f = functools.partial(
      pl.kernel,
      mesh=mesh,
      out_type=jax.ShapeDtypeStruct((NP, OD), jnp.float32),
      scratch_types=[
          pltpu.VMEM((CH, AR), jnp.float32),
          pltpu.VMEM((CH, AR), jnp.float32),
          pltpu.VMEM((IDX_ROWS, 128), jnp.int32),
          pltpu.VMEM((CH * AR, OD), jnp.float32),
          pltpu.VMEM((CH, OD), jnp.float32),
          pltpu.VMEM((AR,), jnp.float32),
          pltpu.SemaphoreType.DMA,
      ],
      compiler_params=pltpu.CompilerParams(use_tc_tiling_on_sc=False, needs_layout_passes=False),
  )(_hoa_body)
  return f(xf, tab)


def kernel(X, params):
  xf = X.reshape(NP, AR)
  tab = params.reshape(D * TAB, OD)
  out = _hoa(xf, tab)
  return out.reshape(B, D, OD)


# speedup vs baseline: 3.0670x; 1.1588x over previous
import functools

import jax
import jax.numpy as jnp
from jax import lax
from jax.experimental import pallas as pl
from jax.experimental.pallas import tpu as pltpu
from jax.experimental.pallas import tpu_sc as plsc

B = 4096
D = 26
AR = 16
OD = 32
TAB = 2 ** AR
NP = B * D
NW = 32
PPW = NP // NW
CH = 64
NCH = PPW // CH
IDX_ROWS = CH * AR // 128


def _hoa_body(x_hbm, tab_hbm, out_hbm, x_v, coef_v, idx_v, rows_v, out_v,
              sem_x, sem_r):
  wid = lax.axis_index("s") * 2 + lax.axis_index("c")
  base = wid * PPW
  iota = lax.iota(jnp.int32, AR)
  prev_idx = jnp.maximum(iota - 1, 0)
  lane0 = iota == 0
  ones = (iota >= 0).astype(jnp.int32)
  zeros_i = iota - iota

  def fire_x(c, slot):
    return pltpu.async_copy(
        x_hbm.at[pl.ds(base + c * CH, CH)], x_v.at[slot], sem_x.at[slot])

  def fire_rows(c, slot):
    return [
        pltpu.async_copy(
            tab_hbm.at[idx_v.at[slot, g]],
            rows_v.at[slot, pl.ds(g * 128, 128)],
            sem_r.at[slot],
        )
        for g in range(IDX_ROWS)
    ]

  fire_x(0, 0)

  def step(c, carry):
    slot = c % 2
    nslot = (c + 1) % 2

    @pl.when(c < NCH)
    def _build():
      pltpu.make_async_copy(
          x_hbm.at[pl.ds(base + c * CH, CH)], x_v.at[slot], sem_x.at[slot]
      ).wait()

      @pl.when(c + 1 < NCH)
      def _():
        fire_x(c + 1, nslot)

      pair0 = base + c * CH

      @plsc.parallel_loop(0, CH, unroll=2)
      def _(p):
        a = x_v[slot, p]
        sa, ind = plsc.sort_key_val(a, iota)
        prev = jnp.take(sa, prev_idx)
        coef = jnp.where(lane0, sa, sa - prev)
        sh = jnp.left_shift(ones, ind)
        cs = plsc.cumsum(sh)
        field = (pair0 + p) % D
        gidx = (field * TAB + 65535 - cs) + sh
        idx_v[slot, p // 8, pl.ds((p % 8) * AR, AR)] = gidx
        coef_v[slot, p] = coef

      fire_rows(c, slot)

    @pl.when(c > 0)
    def _reduce():
      d = c - 1
      dslot = d % 2
      for g in range(IDX_ROWS):
        pltpu.make_async_copy(
            tab_hbm.at[idx_v.at[dslot, g]],
            rows_v.at[dslot, pl.ds(g * 128, 128)],
            sem_r.at[dslot],
        ).wait()

      @plsc.parallel_loop(0, CH, unroll=2)
      def _(p):
        crow = coef_v[dslot, p]
        acc0 = jnp.zeros((16,), jnp.float32)
        acc1 = jnp.zeros((16,), jnp.float32)
        kvec = zeros_i
        r = p * AR
        for k in range(AR):
          cb = jnp.take(crow, kvec)
          kvec = kvec + ones
          acc0 = acc0 + cb * rows_v[dslot, r + k, 0:16]
          acc1 = acc1 + cb * rows_v[dslot, r + k, 16:32]
        out_v[p, 0:16] = acc0
        out_v[p, 16:32] = acc1

      pltpu.sync_copy(out_v, out_hbm.at[pl.ds(base + d * CH, CH)])

    return carry

  lax.fori_loop(0, NCH + 1, step, 0)


@jax.jit
def _hoa(xf, tab):
  mesh = plsc.VectorSubcoreMesh(core_axis_name="c", subcore_axis_name="s")
  f = functools.partial(
      pl.kernel,
      mesh=mesh,
      out_type=jax.ShapeDtypeStruct((NP, OD), jnp.float32),
      scratch_types=[
          pltpu.VMEM((2, CH, AR), jnp.float32),
          pltpu.VMEM((2, CH, AR), jnp.float32),
          pltpu.VMEM((2, IDX_ROWS, 128), jnp.int32),
          pltpu.VMEM((2, CH * AR, OD), jnp.float32),
          pltpu.VMEM((CH, OD), jnp.float32),
          pltpu.SemaphoreType.DMA((2,)),
          pltpu.SemaphoreType.DMA((2,)),
      ],
      compiler_params=pltpu.CompilerParams(
          use_tc_tiling_on_sc=False, needs_layout_passes=False),
  )(_hoa_body)
  return f(xf, tab)


def kernel(X, params):
  xf = X.reshape(NP, AR)
  tab = params.reshape(D * TAB, OD)
  out = _hoa(xf, tab)
  return out.reshape(B, D, OD)
